# baseline (speedup 1.0000x reference)
import functools

import jax
import jax.numpy as jnp
from jax import lax
from jax.experimental import pallas as pl
from jax.experimental.pallas import tpu as pltpu
from jax.experimental.pallas import tpu_sc as plsc

_K = 64
_START = 61
_END = 63
_PADW = 8


def _crf_fwd_kernel(scores_t_ref, gold_ref, t_ref, out_ref, ef_scr, eb_scr):
    T = t_ref[:]
    Kn = T.shape[0]
    s_all = jnp.transpose(scores_t_ref[:], (1, 0, 2))
    Ln, Bn, _ = s_all.shape
    Wn = Kn + _PADW

    expT = jnp.exp(T)
    padc = jnp.zeros((Kn, _PADW - 1), jnp.float32)
    padr = jnp.zeros((_PADW, Wn), jnp.float32)

    def augment(M):
        rs = jnp.sum(M, axis=1, keepdims=True)
        return jnp.concatenate(
            [jnp.concatenate([M, rs, padc], axis=1), padr],
            axis=0).astype(jnp.bfloat16)

    XaF = augment(expT)
    XaB = augment(expT.T)

    exps = jnp.exp(s_all)
    zpad = jnp.zeros((Ln, Bn, _PADW), jnp.float32)
    opad = jnp.concatenate(
        [jnp.ones((Ln, Bn, 1), jnp.float32),
         jnp.zeros((Ln, Bn, _PADW - 1), jnp.float32)], axis=2)
    ef_scr[:] = jnp.concatenate([exps, zpad], axis=2).astype(jnp.bfloat16)
    eb_scr[:] = jnp.concatenate([exps, opad], axis=2).astype(jnp.bfloat16)

    c0 = jnp.max(T[_START, :])
    v0 = jnp.exp(T[_START, :] - c0)
    row_one = (jax.lax.broadcasted_iota(jnp.int32, (Bn, 1), 0)
               .astype(jnp.float32) * 0.0) + 1.0
    ones = jnp.broadcast_to(row_one, (Bn, Wn))
    zer = ones - ones
    a0row = jnp.concatenate(
        [v0, jnp.sum(v0)[None], jnp.zeros((_PADW - 1,), jnp.float32)])
    Af0 = a0row[None, :] * ones
    lane = jax.lax.broadcasted_iota(jnp.int32, (1, Wn), 1)
    u0row = ((lane == _END) | (lane == Kn)).astype(jnp.float32)
    Au0 = u0row * ones

    def body(t, carry):
        Af, rf, Au, ru, loga = carry
        Fv = Af * rf
        Uv = Au * ru
        hf = Fv[:, Kn:Kn + 1]
        hu = Uv[:, Kn:Kn + 1]
        rfn = jnp.broadcast_to(1.0 / hf, (Bn, Wn))
        run = jnp.broadcast_to(1.0 / hu, (Bn, Wn))
        Gf = (Fv * ef_scr[t].astype(jnp.float32)).astype(jnp.bfloat16)
        Afn = jnp.concatenate([
            jax.lax.dot_general(Gf[:16], XaF, (((1,), (0,)), ((), ())),
                                preferred_element_type=jnp.float32),
            jax.lax.dot_general(Gf[16:], XaF, (((1,), (0,)), ((), ())),
                                preferred_element_type=jnp.float32)], axis=0)
        Gu = Uv.astype(jnp.bfloat16)
        Aun = jnp.concatenate([
            jax.lax.dot_general(Gu[:16], XaB, (((1,), (0,)), ((), ())),
                                preferred_element_type=jnp.float32),
            jax.lax.dot_general(Gu[16:], XaB, (((1,), (0,)), ((), ())),
                                preferred_element_type=jnp.float32)], axis=0
        ) * eb_scr[Ln - 1 - t].astype(jnp.float32)
        return Afn, rfn, Aun, run, loga - jnp.log(rfn) - jnp.log(run)

    def body2(i, carry):
        return body(2 * i + 1, body(2 * i, carry))

    Af, rf, Au, ru, loga = jax.lax.fori_loop(
        0, Ln // 4, body2, (Af0, ones, Au0, ones, zer))
    Ff = Af * rf
    Uf = Au * ru
    dot_mid = jnp.sum((Ff * Uf)[:, :Kn], axis=1)
    fs_end = c0 + loga[:, 0] + jnp.log(dot_mid)
    forscores = jnp.sum(fs_end)

    sum_s0 = jnp.sum(s_all[:, :, 0])
    tg_tc = Bn * T[0, _START] + sum_s0

    loss = (forscores - tg_tc) / Bn
    out_ref[:, :] = jnp.broadcast_to(loss, (1, 1))


def _sc_gather_sum(table_rep, gold_flat):
    info = plsc.get_sparse_core_info()
    NC, NS = info.num_cores, info.num_subcores
    NW = NC * NS
    N = gold_flat.shape[0]
    per = N // NW
    mesh = plsc.VectorSubcoreMesh(core_axis_name="c", subcore_axis_name="s")

    @functools.partial(
        pl.kernel, mesh=mesh,
        out_type=jax.ShapeDtypeStruct((NW, 16), jnp.float32),
        scratch_types=[pltpu.VMEM((per,), jnp.int32),
                       pltpu.VMEM((per, 128), jnp.float32),
                       pltpu.VMEM((16,), jnp.float32),
                       pltpu.SemaphoreType.DMA],
    )
    def k(table_hbm, idx_hbm, out_hbm, idx_v, rows_v, acc_v, sem):
        wid = lax.axis_index("s") * NC + lax.axis_index("c")
        base = wid * per
        pltpu.sync_copy(idx_hbm.at[pl.ds(base, per)], idx_v)
        pltpu.async_copy(table_hbm.at[idx_v], rows_v, sem).wait()

        def bd(i, acc):
            return acc + rows_v[i, 0:16]

        acc_v[...] = jax.lax.fori_loop(
            0, per, bd, jnp.zeros((16,), jnp.float32))
        pltpu.sync_copy(acc_v, out_hbm.at[wid])

    return k(table_rep, gold_flat)


def kernel(scores, gold_target, transitions):
    B, L, K = scores.shape
    out = pl.pallas_call(
        _crf_fwd_kernel,
        out_shape=jax.ShapeDtypeStruct((1, 1), jnp.float32),
        scratch_shapes=[pltpu.VMEM((L, B, K + _PADW), jnp.bfloat16),
                        pltpu.VMEM((L, B, K + _PADW), jnp.bfloat16)],
    )(scores, gold_target, transitions)
    NW = 32
    table_rep = jnp.broadcast_to(transitions[0, :, None], (K, 128))
    parts = _sc_gather_sum(table_rep, gold_target.reshape(B * L))
    gather_sum = jnp.sum(parts) / 16.0
    return out[0, 0] - gather_sum / B


# speedup vs baseline: 1.6486x; 1.6486x over previous
import jax
import jax.numpy as jnp
from jax.experimental import pallas as pl
from jax.experimental.pallas import tpu as pltpu

_K = 64
_START = 61
_END = 63
_PADW = 8


def _crf_fwd_kernel(scores_t_ref, gold_ref, t_ref, out_ref, ef_scr, eb_scr):
    T = t_ref[:]
    Kn = T.shape[0]
    s_all = jnp.transpose(scores_t_ref[:], (1, 0, 2))
    Ln, Bn, _ = s_all.shape
    Wn = Kn + _PADW

    expT = jnp.exp(T)
    padc = jnp.zeros((Kn, _PADW - 1), jnp.float32)
    padr = jnp.zeros((_PADW, Wn), jnp.float32)

    def augment(M):
        rs = jnp.sum(M, axis=1, keepdims=True)
        return jnp.concatenate(
            [jnp.concatenate([M, rs, padc], axis=1), padr],
            axis=0).astype(jnp.bfloat16)

    XaF = augment(expT)
    XaB = augment(expT.T)

    exps = jnp.exp(s_all)
    zpad = jnp.zeros((Ln, Bn, _PADW), jnp.float32)
    opad = jnp.concatenate(
        [jnp.ones((Ln, Bn, 1), jnp.float32),
         jnp.zeros((Ln, Bn, _PADW - 1), jnp.float32)], axis=2)
    ef_scr[:] = jnp.concatenate([exps, zpad], axis=2).astype(jnp.bfloat16)
    eb_scr[:] = jnp.concatenate([exps, opad], axis=2).astype(jnp.bfloat16)

    c0 = jnp.max(T[_START, :])
    v0 = jnp.exp(T[_START, :] - c0)
    row_one = (jax.lax.broadcasted_iota(jnp.int32, (Bn, 1), 0)
               .astype(jnp.float32) * 0.0) + 1.0
    ones = jnp.broadcast_to(row_one, (Bn, Wn))
    zer = ones - ones
    a0row = jnp.concatenate(
        [v0, jnp.sum(v0)[None], jnp.zeros((_PADW - 1,), jnp.float32)])
    Af0 = a0row[None, :] * ones
    lane = jax.lax.broadcasted_iota(jnp.int32, (1, Wn), 1)
    u0row = ((lane == _END) | (lane == Kn)).astype(jnp.float32)
    Au0 = u0row * ones

    def body(t, carry):
        Af, rf, Au, ru, loga = carry
        Fv = Af * rf
        Uv = Au * ru
        hf = Fv[:, Kn:Kn + 1]
        hu = Uv[:, Kn:Kn + 1]
        rfn = jnp.broadcast_to(1.0 / hf, (Bn, Wn))
        run = jnp.broadcast_to(1.0 / hu, (Bn, Wn))
        Gf = (Fv * ef_scr[t].astype(jnp.float32)).astype(jnp.bfloat16)
        Afn = jnp.concatenate([
            jax.lax.dot_general(Gf[:16], XaF, (((1,), (0,)), ((), ())),
                                preferred_element_type=jnp.float32),
            jax.lax.dot_general(Gf[16:], XaF, (((1,), (0,)), ((), ())),
                                preferred_element_type=jnp.float32)], axis=0)
        Gu = Uv.astype(jnp.bfloat16)
        Aun = jnp.concatenate([
            jax.lax.dot_general(Gu[:16], XaB, (((1,), (0,)), ((), ())),
                                preferred_element_type=jnp.float32),
            jax.lax.dot_general(Gu[16:], XaB, (((1,), (0,)), ((), ())),
                                preferred_element_type=jnp.float32)], axis=0
        ) * eb_scr[Ln - 1 - t].astype(jnp.float32)
        return Afn, rfn, Aun, run, loga - jnp.log(rfn) - jnp.log(run)

    def body2(i, carry):
        return body(2 * i + 1, body(2 * i, carry))

    Af, rf, Au, ru, loga = jax.lax.fori_loop(
        0, Ln // 4, body2, (Af0, ones, Au0, ones, zer))
    Ff = Af * rf
    Uf = Au * ru
    dot_mid = jnp.sum((Ff * Uf)[:, :Kn], axis=1)
    fs_end = c0 + loga[:, 0] + jnp.log(dot_mid)
    forscores = jnp.sum(fs_end)

    sum_s0 = jnp.sum(s_all[:, :, 0])
    g = gold_ref[:]
    onehot = (g[:, :, None] ==
              jax.lax.broadcasted_iota(jnp.int32, (1, 1, Kn), 2))
    cnt = jnp.sum(onehot.astype(jnp.float32), axis=(0, 1))
    tg = Bn * T[0, _START] + sum_s0 + jnp.sum(cnt * T[0, :])

    loss = (forscores - tg) / Bn
    out_ref[:, :] = jnp.broadcast_to(loss, (1, 1))


def kernel(scores, gold_target, transitions):
    B, L, K = scores.shape
    out = pl.pallas_call(
        _crf_fwd_kernel,
        out_shape=jax.ShapeDtypeStruct((1, 1), jnp.float32),
        scratch_shapes=[pltpu.VMEM((L, B, K + _PADW), jnp.bfloat16),
                        pltpu.VMEM((L, B, K + _PADW), jnp.bfloat16)],
    )(scores, gold_target, transitions)
    return out[0, 0]


# speedup vs baseline: 1.6838x; 1.0213x over previous
import jax
import jax.numpy as jnp
from jax.experimental import pallas as pl
from jax.experimental.pallas import tpu as pltpu

_K = 64
_START = 61
_END = 63
_PADW = 8


def _crf_fwd_kernel(scores_t_ref, gold_ref, t_ref, out_ref, ef_scr, eb_scr):
    T = t_ref[:]
    Kn = T.shape[0]
    s_all = jnp.transpose(scores_t_ref[:], (1, 0, 2))
    Ln, Bn, _ = s_all.shape
    Wn = Kn + _PADW

    expT = jnp.exp(T)
    padc = jnp.zeros((Kn, _PADW - 1), jnp.float32)
    padr = jnp.zeros((_PADW, Wn), jnp.float32)

    def augment(M):
        rs = jnp.sum(M, axis=1, keepdims=True)
        return jnp.concatenate(
            [jnp.concatenate([M, rs, padc], axis=1), padr],
            axis=0).astype(jnp.bfloat16)

    XaF = augment(expT)
    XaB = augment(expT.T)

    exps = jnp.exp(s_all)
    zpad = jnp.zeros((Ln, Bn, _PADW), jnp.float32)
    opad = jnp.concatenate(
        [jnp.ones((Ln, Bn, 1), jnp.float32),
         jnp.zeros((Ln, Bn, _PADW - 1), jnp.float32)], axis=2)
    ef_scr[:] = jnp.concatenate([exps, zpad], axis=2).astype(jnp.bfloat16)
    eb_scr[:] = jnp.concatenate([exps, opad], axis=2).astype(jnp.bfloat16)

    c0 = jnp.max(T[_START, :])
    v0 = jnp.exp(T[_START, :] - c0)
    row_one = (jax.lax.broadcasted_iota(jnp.int32, (Bn, 1), 0)
               .astype(jnp.float32) * 0.0) + 1.0
    ones = jnp.broadcast_to(row_one, (Bn, Wn))
    zer = ones - ones
    a0row = jnp.concatenate(
        [v0, jnp.sum(v0)[None], jnp.zeros((_PADW - 1,), jnp.float32)])
    Af0 = a0row[None, :] * ones
    lane = jax.lax.broadcasted_iota(jnp.int32, (1, Wn), 1)
    u0row = ((lane == _END) | (lane == Kn)).astype(jnp.float32)
    Au0 = u0row * ones

    def body(t, carry):
        Af, rf, Au, ru, loga = carry
        Fv = Af * rf
        Uv = Au * ru
        hf = Fv[:, Kn:Kn + 1]
        hu = Uv[:, Kn:Kn + 1]
        rfn = jnp.broadcast_to(1.0 / hf, (Bn, Wn))
        run = jnp.broadcast_to(1.0 / hu, (Bn, Wn))
        Gf = (Fv * ef_scr[t].astype(jnp.float32)).astype(jnp.bfloat16)
        Afn = jnp.concatenate([
            jax.lax.dot_general(Gf[:16], XaF, (((1,), (0,)), ((), ())),
                                preferred_element_type=jnp.float32),
            jax.lax.dot_general(Gf[16:], XaF, (((1,), (0,)), ((), ())),
                                preferred_element_type=jnp.float32)], axis=0)
        Gu = Uv.astype(jnp.bfloat16)
        Aun = jnp.concatenate([
            jax.lax.dot_general(Gu[:16], XaB, (((1,), (0,)), ((), ())),
                                preferred_element_type=jnp.float32),
            jax.lax.dot_general(Gu[16:], XaB, (((1,), (0,)), ((), ())),
                                preferred_element_type=jnp.float32)], axis=0
        ) * eb_scr[Ln - 1 - t].astype(jnp.float32)
        return Afn, rfn, Aun, run, loga - jnp.log(rfn) - jnp.log(run)

    def body2(i, carry):
        for j in range(4):
            carry = body(4 * i + j, carry)
        return carry

    Af, rf, Au, ru, loga = jax.lax.fori_loop(
        0, Ln // 8, body2, (Af0, ones, Au0, ones, zer))
    Ff = Af * rf
    Uf = Au * ru
    dot_mid = jnp.sum((Ff * Uf)[:, :Kn], axis=1)
    fs_end = c0 + loga[:, 0] + jnp.log(dot_mid)
    forscores = jnp.sum(fs_end)

    sum_s0 = jnp.sum(s_all[:, :, 0])
    g = gold_ref[:]
    onehot = (g[:, :, None] ==
              jax.lax.broadcasted_iota(jnp.int32, (1, 1, Kn), 2))
    cnt = jnp.sum(onehot.astype(jnp.float32), axis=(0, 1))
    tg = Bn * T[0, _START] + sum_s0 + jnp.sum(cnt * T[0, :])

    loss = (forscores - tg) / Bn
    out_ref[:, :] = jnp.broadcast_to(loss, (1, 1))


def kernel(scores, gold_target, transitions):
    B, L, K = scores.shape
    out = pl.pallas_call(
        _crf_fwd_kernel,
        out_shape=jax.ShapeDtypeStruct((1, 1), jnp.float32),
        scratch_shapes=[pltpu.VMEM((L, B, K + _PADW), jnp.bfloat16),
                        pltpu.VMEM((L, B, K + _PADW), jnp.bfloat16)],
    )(scores, gold_target, transitions)
    return out[0, 0]
